# Initial kernel scaffold; baseline (speedup 1.0000x reference)
import jax
import jax.numpy as jnp
from jax.experimental import pallas as pl


def kernel(x, embedding):
    raise NotImplementedError("write your pallas kernel here")


# speedup vs baseline: 7.0072x; 7.0072x over previous
import functools
import jax
import jax.numpy as jnp
from jax import lax
from jax.experimental import pallas as pl
from jax.experimental.pallas import tpu as pltpu, tpu_sc as plsc

DIM = 128
NC, NS = 2, 16
NW = NC * NS
G = 128


def _build(B):
    assert B % (NW * G) == 0
    groups_per_w = B // (NW * G)
    rows_per_w = B // NW

    mesh = plsc.VectorSubcoreMesh(core_axis_name="c", subcore_axis_name="s")

    @functools.partial(
        pl.kernel,
        out_type=jax.ShapeDtypeStruct((B, DIM), jnp.float32),
        mesh=mesh,
        scratch_types=[
            pltpu.VMEM((groups_per_w, G), jnp.int32),
            pltpu.VMEM((G, DIM), jnp.float32),
            pltpu.SemaphoreType.DMA,
        ],
    )
    def k(table_hbm, idx_hbm, out_hbm, idx_v, rows_v, sem):
        wid = lax.axis_index("s") * NC + lax.axis_index("c")
        pltpu.sync_copy(idx_hbm.at[pl.ds(wid * groups_per_w, groups_per_w)], idx_v)
        base = wid * rows_per_w

        @pl.loop(0, groups_per_w)
        def _(j):
            pltpu.async_copy(table_hbm.at[idx_v.at[j]], rows_v, sem).wait()
            pltpu.sync_copy(rows_v, out_hbm.at[pl.ds(base + j * G, G)])

    return k


_kernel_fn = None


def kernel(x, embedding):
    global _kernel_fn
    B = x.size
    if _kernel_fn is None:
        _kernel_fn = _build(B)
    idx = x.reshape(B // G, G).astype(jnp.int32)
    out = _kernel_fn(embedding, idx)
    return out.reshape(x.shape + (DIM,))


# speedup vs baseline: 10.0415x; 1.4330x over previous
import functools
import jax
import jax.numpy as jnp
from jax import lax
from jax.experimental import pallas as pl
from jax.experimental.pallas import tpu as pltpu, tpu_sc as plsc

DIM = 128
NC, NS = 2, 16
NW = NC * NS
G = 128
NBUF = 4


def _build(B):
    assert B % (NW * G * NBUF) == 0
    groups_per_w = B // (NW * G)
    rows_per_w = B // NW
    n_super = groups_per_w // NBUF

    mesh = plsc.VectorSubcoreMesh(core_axis_name="c", subcore_axis_name="s")

    scratch = [pltpu.VMEM((groups_per_w, G), jnp.int32)]
    scratch += [pltpu.VMEM((G, DIM), jnp.float32) for _ in range(NBUF)]
    scratch += [pltpu.SemaphoreType.DMA for _ in range(2 * NBUF)]

    @functools.partial(
        pl.kernel,
        out_type=jax.ShapeDtypeStruct((B, DIM), jnp.float32),
        mesh=mesh,
        scratch_types=scratch,
    )
    def k(table_hbm, idx_hbm, out_hbm, idx_v, *bufs_and_sems):
        rows = bufs_and_sems[:NBUF]
        gsem = bufs_and_sems[NBUF:2 * NBUF]
        ssem = bufs_and_sems[2 * NBUF:]
        wid = lax.axis_index("s") * NC + lax.axis_index("c")
        pltpu.sync_copy(idx_hbm.at[pl.ds(wid * groups_per_w, groups_per_w)], idx_v)
        base = wid * rows_per_w

        for p in range(NBUF):
            pltpu.async_copy(table_hbm.at[idx_v.at[p]], rows[p], gsem[p])

        @pl.loop(0, n_super)
        def _(t):
            j0 = t * NBUF
            for p in range(NBUF):
                pltpu.make_async_copy(
                    table_hbm.at[idx_v.at[j0 + p]], rows[p], gsem[p]).wait()
                pltpu.async_copy(
                    rows[p], out_hbm.at[pl.ds(base + (j0 + p) * G, G)], ssem[p])

            @pl.when(t < n_super - 1)
            def _():
                for p in range(NBUF):
                    pltpu.make_async_copy(
                        rows[p], out_hbm.at[pl.ds(base + (j0 + p) * G, G)],
                        ssem[p]).wait()
                    pltpu.async_copy(
                        table_hbm.at[idx_v.at[j0 + NBUF + p]], rows[p], gsem[p])

        j0 = (n_super - 1) * NBUF
        for p in range(NBUF):
            pltpu.make_async_copy(
                rows[p], out_hbm.at[pl.ds(base + (j0 + p) * G, G)], ssem[p]).wait()

    return k


_kernel_fn = None


def kernel(x, embedding):
    global _kernel_fn
    B = x.size
    if _kernel_fn is None:
        _kernel_fn = _build(B)
    idx = x.reshape(B // G, G).astype(jnp.int32)
    out = _kernel_fn(embedding, idx)
    return out.reshape(x.shape + (DIM,))


# speedup vs baseline: 11.6693x; 1.1621x over previous
import functools
import jax
import jax.numpy as jnp
from jax import lax
from jax.experimental import pallas as pl
from jax.experimental.pallas import tpu as pltpu, tpu_sc as plsc

DIM = 128
VOCAB = 10000
NC, NS = 2, 16
NW = NC * NS
G = 128
NBUF = 2
CHUNK_G = 40


def _build(B):
    groups_per_w = B // (NW * G)
    assert B % (NW * G) == 0 and groups_per_w % CHUNK_G == 0
    n_chunks = groups_per_w // CHUNK_G
    n_super = CHUNK_G // NBUF
    rows_per_w = B // NW

    mesh = plsc.VectorSubcoreMesh(core_axis_name="c", subcore_axis_name="s")

    scratch = [pltpu.VMEM_SHARED((VOCAB, DIM), jnp.float32)]
    scratch += [pltpu.VMEM((CHUNK_G, G), jnp.int32) for _ in range(2)]
    scratch += [pltpu.VMEM((G, DIM), jnp.float32) for _ in range(NBUF)]
    scratch += [pltpu.SemaphoreType.DMA for _ in range(2 + 2 * NBUF)]

    @functools.partial(
        pl.kernel,
        out_type=jax.ShapeDtypeStruct((B, DIM), jnp.float32),
        mesh=mesh,
        scratch_types=scratch,
    )
    def k(table_hbm, idx_hbm, out_hbm, table_sp, *scr):
        idxb = scr[0:2]
        rows = scr[2:2 + NBUF]
        isem = scr[2 + NBUF:4 + NBUF]
        gsem = scr[4 + NBUF:4 + 2 * NBUF]
        ssem = scr[4 + 2 * NBUF:]
        sid = lax.axis_index("s")
        wid = sid * NC + lax.axis_index("c")
        pltpu.sync_copy(table_hbm.at[pl.ds(sid * 624, 624)],
                        table_sp.at[pl.ds(sid * 624, 624)])

        @pl.when(sid == 0)
        def _():
            pltpu.sync_copy(table_hbm.at[pl.ds(9984, 16)],
                            table_sp.at[pl.ds(9984, 16)])

        gbase = wid * groups_per_w
        pltpu.async_copy(idx_hbm.at[pl.ds(gbase, CHUNK_G)], idxb[0], isem[0])
        plsc.subcore_barrier()
        base = wid * rows_per_w
        pltpu.make_async_copy(
            idx_hbm.at[pl.ds(gbase, CHUNK_G)], idxb[0], isem[0]).wait()

        for ci in range(n_chunks):
            idx_v = idxb[ci % 2]
            cbase = base + ci * CHUNK_G * G
            if ci + 1 < n_chunks:
                pltpu.async_copy(
                    idx_hbm.at[pl.ds(gbase + (ci + 1) * CHUNK_G, CHUNK_G)],
                    idxb[(ci + 1) % 2], isem[(ci + 1) % 2])

            for p in range(NBUF):
                pltpu.async_copy(table_sp.at[idx_v.at[p]], rows[p], gsem[p])

            @pl.loop(0, n_super)
            def _(t):
                j0 = t * NBUF
                for p in range(NBUF):
                    pltpu.make_async_copy(
                        table_sp.at[idx_v.at[j0 + p]], rows[p], gsem[p]).wait()
                    pltpu.async_copy(
                        rows[p], out_hbm.at[pl.ds(cbase + (j0 + p) * G, G)],
                        ssem[p])

                @pl.when(t < n_super - 1)
                def _():
                    for p in range(NBUF):
                        pltpu.make_async_copy(
                            rows[p],
                            out_hbm.at[pl.ds(cbase + (j0 + p) * G, G)],
                            ssem[p]).wait()
                        pltpu.async_copy(
                            table_sp.at[idx_v.at[j0 + NBUF + p]], rows[p],
                            gsem[p])

            j0 = (n_super - 1) * NBUF
            for p in range(NBUF):
                pltpu.make_async_copy(
                    rows[p], out_hbm.at[pl.ds(cbase + (j0 + p) * G, G)],
                    ssem[p]).wait()
            if ci + 1 < n_chunks:
                pltpu.make_async_copy(
                    idx_hbm.at[pl.ds(gbase + (ci + 1) * CHUNK_G, CHUNK_G)],
                    idxb[(ci + 1) % 2], isem[(ci + 1) % 2]).wait()

    return k


_kernel_fn = None


def kernel(x, embedding):
    global _kernel_fn
    B = x.size
    if _kernel_fn is None:
        _kernel_fn = _build(B)
    idx = x.reshape(B // G, G).astype(jnp.int32)
    out = _kernel_fn(embedding, idx)
    return out.reshape(x.shape + (DIM,))


# speedup vs baseline: 11.7229x; 1.0046x over previous
import functools
import jax
import jax.numpy as jnp
from jax import lax
from jax.experimental import pallas as pl
from jax.experimental.pallas import tpu as pltpu, tpu_sc as plsc

DIM = 128
VOCAB = 10000
NC, NS = 2, 16
NW = NC * NS
G = 128
NBUF = 2


def _build(B):
    groups_per_w = B // (NW * G)
    assert B % (NW * G) == 0 and groups_per_w % NBUF == 0
    n_super = groups_per_w // NBUF
    rows_per_w = B // NW

    mesh = plsc.VectorSubcoreMesh(core_axis_name="c", subcore_axis_name="s")

    scratch = [pltpu.VMEM_SHARED((VOCAB, DIM), jnp.float32),
               pltpu.VMEM((2, NBUF, G), jnp.int32)]
    scratch += [pltpu.VMEM((G, DIM), jnp.float32) for _ in range(NBUF)]
    scratch += [pltpu.SemaphoreType.DMA for _ in range(1 + 2 * NBUF)]

    @functools.partial(
        pl.kernel,
        out_type=jax.ShapeDtypeStruct((B, DIM), jnp.float32),
        mesh=mesh,
        scratch_types=scratch,
    )
    def k(table_hbm, idx_hbm, out_hbm, table_sp, idxb, *scr):
        rows = scr[:NBUF]
        isem = scr[NBUF]
        gsem = scr[NBUF + 1:NBUF + 1 + NBUF]
        ssem = scr[NBUF + 1 + NBUF:]
        sid = lax.axis_index("s")
        wid = sid * NC + lax.axis_index("c")
        pltpu.sync_copy(table_hbm.at[pl.ds(sid * 624, 624)],
                        table_sp.at[pl.ds(sid * 624, 624)])

        @pl.when(sid == 0)
        def _():
            pltpu.sync_copy(table_hbm.at[pl.ds(9984, 16)],
                            table_sp.at[pl.ds(9984, 16)])

        gbase = wid * groups_per_w
        pltpu.sync_copy(idx_hbm.at[pl.ds(gbase, NBUF)], idxb.at[0])
        pltpu.async_copy(idx_hbm.at[pl.ds(gbase + NBUF, NBUF)], idxb.at[1],
                         isem)
        plsc.subcore_barrier()
        base = wid * rows_per_w

        for p in range(NBUF):
            pltpu.async_copy(table_sp.at[idxb.at[0, p]], rows[p], gsem[p])

        @pl.loop(0, n_super)
        def _(t):
            j0 = t * NBUF
            for p in range(NBUF):
                pltpu.make_async_copy(
                    table_sp.at[idxb.at[t % 2, p]], rows[p], gsem[p]).wait()
                pltpu.async_copy(
                    rows[p], out_hbm.at[pl.ds(base + (j0 + p) * G, G)],
                    ssem[p])

            @pl.when(t < n_super - 1)
            def _():
                pltpu.make_async_copy(
                    idx_hbm.at[pl.ds(gbase + (t + 1) * NBUF, NBUF)],
                    idxb.at[(t + 1) % 2], isem).wait()

                @pl.when(t < n_super - 2)
                def _():
                    pltpu.async_copy(
                        idx_hbm.at[pl.ds(gbase + (t + 2) * NBUF, NBUF)],
                        idxb.at[t % 2], isem)

                for p in range(NBUF):
                    pltpu.make_async_copy(
                        rows[p], out_hbm.at[pl.ds(base + (j0 + p) * G, G)],
                        ssem[p]).wait()
                    pltpu.async_copy(
                        table_sp.at[idxb.at[(t + 1) % 2, p]], rows[p],
                        gsem[p])

        j0 = (n_super - 1) * NBUF
        for p in range(NBUF):
            pltpu.make_async_copy(
                rows[p], out_hbm.at[pl.ds(base + (j0 + p) * G, G)],
                ssem[p]).wait()

    return k


_kernel_fn = None


def kernel(x, embedding):
    global _kernel_fn
    B = x.size
    if _kernel_fn is None:
        _kernel_fn = _build(B)
    idx = x.reshape(B // G, G).astype(jnp.int32)
    out = _kernel_fn(embedding, idx)
    return out.reshape(x.shape + (DIM,))


# speedup vs baseline: 16.9356x; 1.4447x over previous
import functools
import jax
import jax.numpy as jnp
from jax import lax
from jax.experimental import pallas as pl
from jax.experimental.pallas import tpu as pltpu, tpu_sc as plsc

DIM = 128
VOCAB = 10000
NC, NS = 2, 16
NW = NC * NS
G = 128
NBUF = 3


def _build(B):
    groups_per_w = B // (NW * G)
    assert B % (NW * G) == 0
    n_super = groups_per_w // NBUF
    n_tail = groups_per_w - n_super * NBUF
    rows_per_w = B // NW
    SS = NBUF * G

    mesh = plsc.VectorSubcoreMesh(core_axis_name="c", subcore_axis_name="s")

    scratch = [pltpu.VMEM_SHARED((VOCAB, DIM), jnp.float32),
               pltpu.VMEM((2, SS), jnp.int32),
               pltpu.VMEM((max(n_tail, 1) * G,), jnp.int32)]
    scratch += [pltpu.VMEM((G, DIM), jnp.float32) for _ in range(NBUF)]
    scratch += [pltpu.SemaphoreType.DMA for _ in range(2 + 2 * NBUF)]

    @functools.partial(
        pl.kernel,
        out_type=jax.ShapeDtypeStruct((B, DIM), jnp.float32),
        mesh=mesh,
        scratch_types=scratch,
    )
    def k(table_hbm, idx_hbm, out_hbm, table_sp, idxb, idxt, *scr):
        rows = scr[:NBUF]
        isem = scr[NBUF]
        tsem = scr[NBUF + 1]
        gsem = scr[NBUF + 2:NBUF + 2 + NBUF]
        ssem = scr[NBUF + 2 + NBUF:]
        sid = lax.axis_index("s")
        wid = sid * NC + lax.axis_index("c")
        pltpu.sync_copy(table_hbm.at[pl.ds(sid * 624, 624)],
                        table_sp.at[pl.ds(sid * 624, 624)])

        @pl.when(sid == 0)
        def _():
            pltpu.sync_copy(table_hbm.at[pl.ds(9984, 16)],
                            table_sp.at[pl.ds(9984, 16)])

        ibase = wid * rows_per_w
        pltpu.sync_copy(idx_hbm.at[pl.ds(ibase, SS)], idxb.at[0])
        pltpu.async_copy(idx_hbm.at[pl.ds(ibase + SS, SS)], idxb.at[1], isem)
        if n_tail:
            pltpu.async_copy(
                idx_hbm.at[pl.ds(ibase + n_super * SS, n_tail * G)], idxt,
                tsem)
        plsc.subcore_barrier()
        base = wid * rows_per_w

        for p in range(NBUF):
            pltpu.async_copy(
                table_sp.at[idxb.at[0, pl.ds(p * G, G)]], rows[p], gsem[p])

        @pl.loop(0, n_super)
        def _(t):
            j0 = t * NBUF
            for p in range(NBUF):
                pltpu.make_async_copy(
                    table_sp.at[idxb.at[t % 2, pl.ds(p * G, G)]], rows[p],
                    gsem[p]).wait()
                pltpu.async_copy(
                    rows[p], out_hbm.at[pl.ds(base + (j0 + p) * G, G)],
                    ssem[p])

            @pl.when(t < n_super - 1)
            def _():
                pltpu.make_async_copy(
                    idx_hbm.at[pl.ds(ibase + (t + 1) * SS, SS)],
                    idxb.at[(t + 1) % 2], isem).wait()

                @pl.when(t < n_super - 2)
                def _():
                    pltpu.async_copy(
                        idx_hbm.at[pl.ds(ibase + (t + 2) * SS, SS)],
                        idxb.at[t % 2], isem)

                for p in range(NBUF):
                    pltpu.make_async_copy(
                        rows[p], out_hbm.at[pl.ds(base + (j0 + p) * G, G)],
                        ssem[p]).wait()
                    pltpu.async_copy(
                        table_sp.at[idxb.at[(t + 1) % 2, pl.ds(p * G, G)]],
                        rows[p], gsem[p])

        j0 = (n_super - 1) * NBUF
        if n_tail:
            pltpu.make_async_copy(
                idx_hbm.at[pl.ds(ibase + n_super * SS, n_tail * G)], idxt,
                tsem).wait()
            for p in range(n_tail):
                pltpu.make_async_copy(
                    rows[p], out_hbm.at[pl.ds(base + (j0 + p) * G, G)],
                    ssem[p]).wait()
                pltpu.async_copy(
                    table_sp.at[idxt.at[pl.ds(p * G, G)]], rows[p], gsem[p])
            for p in range(n_tail):
                jt = n_super * NBUF + p
                pltpu.make_async_copy(
                    table_sp.at[idxt.at[pl.ds(p * G, G)]], rows[p],
                    gsem[p]).wait()
                pltpu.async_copy(
                    rows[p], out_hbm.at[pl.ds(base + jt * G, G)], ssem[p])
            for p in range(n_tail):
                jt = n_super * NBUF + p
                pltpu.make_async_copy(
                    rows[p], out_hbm.at[pl.ds(base + jt * G, G)],
                    ssem[p]).wait()
            for p in range(n_tail, NBUF):
                pltpu.make_async_copy(
                    rows[p], out_hbm.at[pl.ds(base + (j0 + p) * G, G)],
                    ssem[p]).wait()
        else:
            for p in range(NBUF):
                pltpu.make_async_copy(
                    rows[p], out_hbm.at[pl.ds(base + (j0 + p) * G, G)],
                    ssem[p]).wait()

    return k


_kernel_fn = None


def kernel(x, embedding):
    global _kernel_fn
    B = x.size
    if _kernel_fn is None:
        _kernel_fn = _build(B)
    idx = x.reshape(B).astype(jnp.int32)
    out = _kernel_fn(embedding, idx)
    return out.reshape(x.shape + (DIM,))
